# Initial kernel scaffold; baseline (speedup 1.0000x reference)
import jax
import jax.numpy as jnp
from jax.experimental import pallas as pl


def kernel(x, idx):
    raise NotImplementedError("write your pallas kernel here")


# speedup vs baseline: 2.6626x; 2.6626x over previous
import functools

import jax
import jax.numpy as jnp
from jax import lax
from jax.experimental import pallas as pl
from jax.experimental.pallas import tpu as pltpu
from jax.experimental.pallas import tpu_sc as plsc

_N_NODES = 10000
_D = 256
_B = 160000
_NC = 2
_NS = 16
_NW = _NC * _NS
_C = 128
_NCHUNKS = _B // _C
_KMAX = -(-_NCHUNKS // _NW)


def _gather_body(x_hbm, idx_hbm, out_hbm, idx_v, rows_v, sem):
    wid = lax.axis_index("s") * _NC + lax.axis_index("c")

    def body(k, carry):
        chunk = wid + k * _NW

        @pl.when(chunk < _NCHUNKS)
        def _():
            base = chunk * _C
            pltpu.sync_copy(idx_hbm.at[pl.ds(base, _C)], idx_v)
            pltpu.async_copy(x_hbm.at[idx_v], rows_v, sem).wait()
            pltpu.sync_copy(rows_v, out_hbm.at[pl.ds(base, _C)])

        return carry

    lax.fori_loop(0, _KMAX, body, None)


@jax.jit
def _run(x, idx):
    mesh = plsc.VectorSubcoreMesh(core_axis_name="c", subcore_axis_name="s")
    f = pl.kernel(
        _gather_body,
        mesh=mesh,
        out_type=jax.ShapeDtypeStruct((_B, _D), jnp.float32),
        scratch_types=[
            pltpu.VMEM((_C,), jnp.int32),
            pltpu.VMEM((_C, _D), jnp.float32),
            pltpu.SemaphoreType.DMA,
        ],
    )
    return f(x, idx)


def kernel(x, idx):
    return _run(x, idx)


# speedup vs baseline: 3.5407x; 1.3298x over previous
import functools

import jax
import jax.numpy as jnp
from jax import lax
from jax.experimental import pallas as pl
from jax.experimental.pallas import tpu as pltpu
from jax.experimental.pallas import tpu_sc as plsc

_N_NODES = 10000
_D = 256
_B = 160000
_NC = 2
_NS = 16
_NW = _NC * _NS
_C = 128
_NCHUNKS = _B // _C
_KMAX = -(-_NCHUNKS // _NW)
_KC = _KMAX * _C


def _gather_body(x_hbm, idx_hbm, out_hbm, idx_all, rows, g0, g1, w0, w1):
    wid = lax.axis_index("s") * _NC + lax.axis_index("c")
    first_chunk = wid * _KMAX
    load_base = jnp.minimum(first_chunk * _C, _B - _KC)
    pltpu.sync_copy(idx_hbm.at[pl.ds(load_base, _KC)], idx_all)

    gsem = (g0, g1)
    wsem = (w0, w1)

    def valid(k):
        return jnp.logical_and(
            jnp.logical_and(k >= 0, k < _KMAX), first_chunk + k < _NCHUNKS
        )

    def gather_desc(k, b):
        off = (first_chunk + k) * _C - load_base
        return pltpu.make_async_copy(
            x_hbm.at[idx_all.at[pl.ds(off, _C)]], rows.at[b], gsem[b]
        )

    def write_desc(k, b):
        base = (first_chunk + k) * _C
        return pltpu.make_async_copy(
            rows.at[b], out_hbm.at[pl.ds(base, _C)], wsem[b]
        )

    def start_gather(k, b):
        @pl.when(valid(k))
        def _():
            gather_desc(k, b).start()

    def wait_gather(k, b):
        @pl.when(valid(k))
        def _():
            gather_desc(k, b).wait()

    def start_write(k, b):
        @pl.when(valid(k))
        def _():
            write_desc(k, b).start()

    def wait_write(k, b):
        @pl.when(valid(k))
        def _():
            write_desc(k, b).wait()

    start_gather(0, 0)

    def body(k2, carry):
        for b in (0, 1):
            k = k2 * 2 + b
            wait_write(k - 1, 1 - b)
            start_gather(k + 1, 1 - b)
            wait_gather(k, b)
            start_write(k, b)
        return carry

    lax.fori_loop(0, _KMAX // 2, body, None)
    wait_write(_KMAX - 1, 1)


@jax.jit
def _run(x, idx):
    mesh = plsc.VectorSubcoreMesh(core_axis_name="c", subcore_axis_name="s")
    f = pl.kernel(
        _gather_body,
        mesh=mesh,
        out_type=jax.ShapeDtypeStruct((_B, _D), jnp.float32),
        scratch_types=[
            pltpu.VMEM((_KC,), jnp.int32),
            pltpu.VMEM((2, _C, _D), jnp.float32),
            pltpu.SemaphoreType.DMA,
            pltpu.SemaphoreType.DMA,
            pltpu.SemaphoreType.DMA,
            pltpu.SemaphoreType.DMA,
        ],
    )
    return f(x, idx)


def kernel(x, idx):
    return _run(x, idx)
